# baseline (speedup 1.0000x reference)
import jax
import jax.numpy as jnp
from jax import lax
from jax.experimental import pallas as pl
from jax.experimental.pallas import tpu as pltpu
from jax.experimental.pallas import tpu_sc as plsc

N = 100000
D = 128
G = 512
T = 12

NC = 2
NS = 16
NW = NC * NS
L = 16

CH = 128
NCH = (N // CH)
NSTREAM = NW - 2
KPW = 26
LAST_BASE = 780 * CH
TAIL_BASE = NCH * CH
TAIL_ROWS = N - TAIL_BASE
NBUF = 3
BSTEPS = 17
NPACK = N // 2


def _zero_rows(ref, row0, rows):
    z = jnp.zeros((L,), jnp.float32)

    def body(i, carry):
        for j in range(D // L):
            ref[i, pl.ds(j * L, L)] = z
        return carry

    lax.fori_loop(row0, row0 + rows, body, 0)


def _lb_packed(pv, targets):
    half = jnp.full((L,), NPACK, jnp.int32)
    lo = jnp.zeros((L,), jnp.int32)
    hi = jnp.full((L,), N, jnp.int32)
    nm1 = jnp.full((L,), N - 1, jnp.int32)
    one = jnp.full((L,), 1, jnp.int32)
    for _ in range(BSTEPS):
        mid = lax.shift_right_logical(lo + hi, one)
        midc = jnp.minimum(mid, nm1)
        in_lo = midc < half
        word = jnp.where(in_lo, midc, midc - NPACK)
        w = plsc.load_gather(pv, [word])
        sh = jnp.where(in_lo, jnp.zeros((L,), jnp.int32),
                       jnp.full((L,), 16, jnp.int32))
        val = jnp.bitwise_and(lax.shift_right_logical(w, sh),
                              jnp.full((L,), 0xFFFF, jnp.int32))
        pred = val >= targets
        act = lo < hi
        hi = jnp.where(jnp.logical_and(pred, act), midc, hi)
        lo = jnp.where(jnp.logical_and(jnp.logical_not(pred), act),
                       midc + 1, lo)
    return lo


def _sc_segment_sums(x, batch, packed):
    mesh = plsc.VectorSubcoreMesh(core_axis_name="c", subcore_axis_name="s")

    def body(x_hbm, batch_hbm, packed_hbm, parts_hbm, cnts_hbm,
             xbuf0, xbuf1, xbuf2, idxb0, idxb1, idxb2, packed_v, cnt_v,
             semx0, semx1, semx2, semi0, semi1, semi2,
             sems0, sems1, sems2, acc_sh):
        cid = lax.axis_index("c")
        sid = lax.axis_index("s")
        wid = sid * NC + cid
        xbuf = (xbuf0, xbuf1, xbuf2)
        idxb = (idxb0, idxb1, idxb2)
        semx = (semx0, semx1, semx2)
        semi = (semi0, semi1, semi2)
        sems = (sems0, sems1, sems2)

        rows_per_tile = G // NS
        _zero_rows(xbuf0, 0, rows_per_tile)
        pltpu.sync_copy(xbuf0.at[pl.ds(0, rows_per_tile)],
                        acc_sh.at[pl.ds(sid * rows_per_tile, rows_per_tile)])
        plsc.subcore_barrier()

        @pl.when(wid == 1)
        def _():
            pltpu.sync_copy(x_hbm.at[pl.ds(LAST_BASE, CH)], xbuf0)
            pltpu.sync_copy(batch_hbm.at[pl.ds(LAST_BASE, CH)], idxb0.at[0])
            pltpu.sync_copy(xbuf0, acc_sh.at[idxb0.at[0]], add=True)
            _zero_rows(xbuf1, TAIL_ROWS, CH - TAIL_ROWS)
            zi = jnp.zeros((L,), jnp.int32)
            for j in range(TAIL_ROWS // L, CH // L):
                idxb1[0, pl.ds(j * L, L)] = zi
            pltpu.sync_copy(x_hbm.at[pl.ds(TAIL_BASE, TAIL_ROWS)],
                            xbuf1.at[pl.ds(0, TAIL_ROWS)])
            pltpu.sync_copy(batch_hbm.at[pl.ds(TAIL_BASE, TAIL_ROWS)],
                            idxb1.at[0, pl.ds(0, TAIL_ROWS)])
            pltpu.sync_copy(xbuf1, acc_sh.at[idxb1.at[0]], add=True)

        @pl.when(wid < 2)
        def _():
            pltpu.sync_copy(packed_hbm, packed_v)
            lane = lax.broadcasted_iota(jnp.int32, (L,), 0)
            half = wid * (G // 2)

            def cnt_body(t, carry):
                g0 = half + t * L
                lb_lo = _lb_packed(packed_v, g0 + lane)
                lb_hi = _lb_packed(packed_v, g0 + 1 + lane)
                cnt_v[pl.ds(g0, L)] = (lb_hi - lb_lo).astype(jnp.float32)
                return carry

            lax.fori_loop(0, G // 2 // L, cnt_body, 0)
            pltpu.sync_copy(cnt_v.at[pl.ds(half, G // 2)],
                            cnts_hbm.at[pl.ds(half, G // 2)])

        @pl.when(wid >= 2)
        def _():
            j = wid - 2

            def issue(k):
                b = k % NBUF
                c = j + NSTREAM * k
                di = pltpu.async_copy(batch_hbm.at[pl.ds(c * CH, CH)],
                                      idxb[b].at[0], semi[b])
                dx = pltpu.async_copy(x_hbm.at[pl.ds(c * CH, CH)], xbuf[b],
                                      semx[b])
                return di, dx

            dd = {0: issue(0), 1: issue(1)}
            ss = {}
            for k in range(KPW):
                b = k % NBUF
                di, dx = dd.pop(k)
                di.wait()
                dx.wait()
                ss[k] = pltpu.async_copy(xbuf[b], acc_sh.at[idxb[b].at[0]],
                                         sems[b], add=True)
                if k + 2 < KPW:
                    if k - 1 >= 0:
                        ss.pop(k - 1).wait()
                    dd[k + 2] = issue(k + 2)
            for k in sorted(ss):
                ss[k].wait()

        plsc.subcore_barrier()

        lo = sid * rows_per_tile
        pltpu.sync_copy(acc_sh.at[pl.ds(lo, rows_per_tile)],
                        parts_hbm.at[cid, pl.ds(lo, rows_per_tile)])

    return pl.kernel(
        body,
        out_type=(
            jax.ShapeDtypeStruct((NC, G, D), jnp.float32),
            jax.ShapeDtypeStruct((G,), jnp.float32),
        ),
        mesh=mesh,
        scratch_types=(
            [pltpu.VMEM((CH, D), jnp.float32) for _ in range(NBUF)]
            + [pltpu.VMEM((1, CH), jnp.int32) for _ in range(NBUF)]
            + [
                pltpu.VMEM((NPACK,), jnp.int32),
                pltpu.VMEM((G,), jnp.float32),
            ]
            + [pltpu.SemaphoreType.DMA for _ in range(3 * NBUF)]
            + [pltpu.VMEM_SHARED((G, D), jnp.float32)]
        ),
        compiler_params=pltpu.CompilerParams(needs_layout_passes=False),
    )(x, batch, packed)


def _head_body(parts_ref, cnts_ref, w_ref, b_ref, o_ref):
    sums = parts_ref[0] + parts_ref[1]
    rep = sums / jnp.maximum(cnts_ref[...], 1.0)
    o_ref[...] = (
        jnp.dot(rep, w_ref[...], preferred_element_type=jnp.float32)
        + b_ref[...]
    )


def kernel(x, batch, W, b):
    bi = batch.astype(jnp.int32)
    packed = jnp.bitwise_or(bi[:NPACK], jnp.left_shift(bi[NPACK:], 16))
    parts, cnts = _sc_segment_sums(x, bi, packed)
    out = pl.pallas_call(
        _head_body,
        out_shape=jax.ShapeDtypeStruct((G, T), jnp.float32),
    )(parts, cnts.reshape(G, 1), W, b.reshape(1, T))
    return out


# speedup vs baseline: 1.0089x; 1.0089x over previous
import jax
import jax.numpy as jnp
from jax import lax
from jax.experimental import pallas as pl
from jax.experimental.pallas import tpu as pltpu
from jax.experimental.pallas import tpu_sc as plsc

N = 100000
D = 128
G = 512
T = 12

NC = 2
NS = 16
NW = NC * NS
L = 16

CH = 256
NSTREAM = NW - 2
KPW = 10
SC_ROWS = CH * NSTREAM * KPW
TC_ROWS = N - SC_ROWS
TB = 800
TC_BLOCKS = TC_ROWS // TB
BSTEPS = 17
NPACK = N // 2


def _zero_rows(ref, row0, rows):
    z = jnp.zeros((L,), jnp.float32)

    def body(i, carry):
        for j in range(D // L):
            ref[i, pl.ds(j * L, L)] = z
        return carry

    lax.fori_loop(row0, row0 + rows, body, 0)


def _lb_packed(pv, targets):
    half = jnp.full((L,), NPACK, jnp.int32)
    lo = jnp.zeros((L,), jnp.int32)
    hi = jnp.full((L,), N, jnp.int32)
    nm1 = jnp.full((L,), N - 1, jnp.int32)
    one = jnp.full((L,), 1, jnp.int32)
    for _ in range(BSTEPS):
        mid = lax.shift_right_logical(lo + hi, one)
        midc = jnp.minimum(mid, nm1)
        in_lo = midc < half
        word = jnp.where(in_lo, midc, midc - NPACK)
        w = plsc.load_gather(pv, [word])
        sh = jnp.where(in_lo, jnp.zeros((L,), jnp.int32),
                       jnp.full((L,), 16, jnp.int32))
        val = jnp.bitwise_and(lax.shift_right_logical(w, sh),
                              jnp.full((L,), 0xFFFF, jnp.int32))
        pred = val >= targets
        act = lo < hi
        hi = jnp.where(jnp.logical_and(pred, act), midc, hi)
        lo = jnp.where(jnp.logical_and(jnp.logical_not(pred), act),
                       midc + 1, lo)
    return lo


def _sc_segment_sums(x, batch, packed):
    mesh = plsc.VectorSubcoreMesh(core_axis_name="c", subcore_axis_name="s")

    def body(x_hbm, batch_hbm, packed_hbm, parts_hbm, cnts_hbm,
             xbuf0, xbuf1, idxb0, idxb1, packed_v, cnt_v,
             semx0, semx1, semi0, semi1, acc_sh):
        cid = lax.axis_index("c")
        sid = lax.axis_index("s")
        wid = sid * NC + cid
        xbuf = (xbuf0, xbuf1)
        idxb = (idxb0, idxb1)
        semx = (semx0, semx1)
        semi = (semi0, semi1)

        rows_per_tile = G // NS
        _zero_rows(xbuf0, 0, rows_per_tile)
        pltpu.sync_copy(xbuf0.at[pl.ds(0, rows_per_tile)],
                        acc_sh.at[pl.ds(sid * rows_per_tile, rows_per_tile)])
        plsc.subcore_barrier()

        @pl.when(wid < 2)
        def _():
            pltpu.sync_copy(packed_hbm, packed_v)
            lane = lax.broadcasted_iota(jnp.int32, (L,), 0)
            half = wid * (G // 2)

            def cnt_body(t, carry):
                g0 = half + t * L
                lb_lo = _lb_packed(packed_v, g0 + lane)
                lb_hi = _lb_packed(packed_v, g0 + 1 + lane)
                cnt_v[pl.ds(g0, L)] = (lb_hi - lb_lo).astype(jnp.float32)
                return carry

            lax.fori_loop(0, G // 2 // L, cnt_body, 0)
            pltpu.sync_copy(cnt_v.at[pl.ds(half, G // 2)],
                            cnts_hbm.at[pl.ds(half, G // 2)])

        @pl.when(wid >= 2)
        def _():
            j = wid - 2

            def issue(k, b):
                c = j + NSTREAM * k
                di0 = pltpu.async_copy(batch_hbm.at[pl.ds(c * CH, 128)],
                                       idxb[b].at[0], semi[b])
                di1 = pltpu.async_copy(batch_hbm.at[pl.ds(c * CH + 128, 128)],
                                       idxb[b].at[1], semi[b])
                dx = pltpu.async_copy(x_hbm.at[pl.ds(c * CH, CH)], xbuf[b],
                                      semx[b])
                return di0, di1, dx

            descs = {0: issue(0, 0), 1: issue(1, 1)}
            for k in range(KPW):
                b = k & 1
                di0, di1, dx = descs.pop(k)
                di0.wait()
                di1.wait()
                dx.wait()
                for h in range(2):
                    pltpu.sync_copy(xbuf[b].at[pl.ds(h * 128, 128)],
                                    acc_sh.at[idxb[b].at[h]], add=True)
                if k + 2 < KPW:
                    descs[k + 2] = issue(k + 2, b)

        plsc.subcore_barrier()

        lo = sid * rows_per_tile
        pltpu.sync_copy(acc_sh.at[pl.ds(lo, rows_per_tile)],
                        parts_hbm.at[cid, pl.ds(lo, rows_per_tile)])

    return pl.kernel(
        body,
        out_type=(
            jax.ShapeDtypeStruct((NC, G, D), jnp.float32),
            jax.ShapeDtypeStruct((G,), jnp.float32),
        ),
        mesh=mesh,
        scratch_types=[
            pltpu.VMEM((CH, D), jnp.float32),
            pltpu.VMEM((CH, D), jnp.float32),
            pltpu.VMEM((2, 128), jnp.int32),
            pltpu.VMEM((2, 128), jnp.int32),
            pltpu.VMEM((NPACK,), jnp.int32),
            pltpu.VMEM((G,), jnp.float32),
            pltpu.SemaphoreType.DMA,
            pltpu.SemaphoreType.DMA,
            pltpu.SemaphoreType.DMA,
            pltpu.SemaphoreType.DMA,
            pltpu.VMEM_SHARED((G, D), jnp.float32),
        ],
        compiler_params=pltpu.CompilerParams(needs_layout_passes=False),
    )(x, batch, packed)


def _tc_partial_body(xb_ref, bb_ref, o_ref):
    i = pl.program_id(0)

    @pl.when(i == 0)
    def _():
        o_ref[...] = jnp.zeros((G, D), jnp.float32)

    bb = bb_ref[0, 0, :]
    gids = lax.broadcasted_iota(jnp.int32, (G, TB), 0)
    oh = jnp.where(gids == bb[None, :], 1.0, 0.0).astype(jnp.float32)
    o_ref[...] += jnp.dot(oh, xb_ref[...], preferred_element_type=jnp.float32)


def _tc_partial(x_tc, batch_tc):
    return pl.pallas_call(
        _tc_partial_body,
        grid=(TC_BLOCKS,),
        in_specs=[
            pl.BlockSpec((TB, D), lambda i: (i, 0)),
            pl.BlockSpec((1, 1, TB), lambda i: (i, 0, 0)),
        ],
        out_specs=pl.BlockSpec((G, D), lambda i: (0, 0)),
        out_shape=jax.ShapeDtypeStruct((G, D), jnp.float32),
    )(x_tc, batch_tc)


def _head_body(parts_ref, tc_ref, cnts_ref, w_ref, b_ref, o_ref):
    sums = parts_ref[0] + parts_ref[1] + tc_ref[...]
    rep = sums / jnp.maximum(cnts_ref[...], 1.0)
    o_ref[...] = (
        jnp.dot(rep, w_ref[...], preferred_element_type=jnp.float32)
        + b_ref[...]
    )


def kernel(x, batch, W, b):
    bi = batch.astype(jnp.int32)
    packed = jnp.bitwise_or(bi[:NPACK], jnp.left_shift(bi[NPACK:], 16))
    batch_tc = bi[SC_ROWS:].reshape(TC_BLOCKS, 1, TB)
    tc_part = _tc_partial(x[SC_ROWS:], batch_tc)
    parts, cnts = _sc_segment_sums(x, bi, packed)
    out = pl.pallas_call(
        _head_body,
        out_shape=jax.ShapeDtypeStruct((G, T), jnp.float32),
    )(parts, tc_part, cnts.reshape(G, 1), W, b.reshape(1, T))
    return out


# speedup vs baseline: 1.1817x; 1.1712x over previous
import jax
import jax.numpy as jnp
from jax import lax
from jax.experimental import pallas as pl
from jax.experimental.pallas import tpu as pltpu
from jax.experimental.pallas import tpu_sc as plsc

N = 100000
D = 128
G = 512
T = 12

NC = 2
NS = 16
NW = NC * NS
L = 16

CH = 256
NSTREAM = NW - 2
KPW = 10
SC_ROWS = CH * NSTREAM * KPW
TC_ROWS = N - SC_ROWS
TB = 800
TC_BLOCKS = TC_ROWS // TB
BSTEPS = 17
NPACK = N // 2


def _zero_rows(ref, row0, rows):
    z = jnp.zeros((L,), jnp.float32)

    def body(i, carry):
        for j in range(D // L):
            ref[i, pl.ds(j * L, L)] = z
        return carry

    lax.fori_loop(row0, row0 + rows, body, 0)


def _lb_packed(pv, targets):
    half = jnp.full((L,), NPACK, jnp.int32)
    lo = jnp.zeros((L,), jnp.int32)
    hi = jnp.full((L,), N, jnp.int32)
    nm1 = jnp.full((L,), N - 1, jnp.int32)
    one = jnp.full((L,), 1, jnp.int32)
    for _ in range(BSTEPS):
        mid = lax.shift_right_logical(lo + hi, one)
        midc = jnp.minimum(mid, nm1)
        in_lo = midc < half
        word = jnp.where(in_lo, midc, midc - NPACK)
        w = plsc.load_gather(pv, [word])
        sh = jnp.where(in_lo, jnp.zeros((L,), jnp.int32),
                       jnp.full((L,), 16, jnp.int32))
        val = jnp.bitwise_and(lax.shift_right_logical(w, sh),
                              jnp.full((L,), 0xFFFF, jnp.int32))
        pred = val >= targets
        act = lo < hi
        hi = jnp.where(jnp.logical_and(pred, act), midc, hi)
        lo = jnp.where(jnp.logical_and(jnp.logical_not(pred), act),
                       midc + 1, lo)
    return lo


def _sc_segment_sums(x, batch, packed):
    mesh = plsc.VectorSubcoreMesh(core_axis_name="c", subcore_axis_name="s")

    def body(x_hbm, batch_hbm, packed_hbm, parts_hbm, cnts_hbm,
             xbuf0, xbuf1, idxb0, idxb1, packed_v, cnt_v,
             semx0, semx1, semi0, semi1, acc_sh):
        cid = lax.axis_index("c")
        sid = lax.axis_index("s")
        wid = sid * NC + cid
        xbuf = (xbuf0, xbuf1)
        idxb = (idxb0, idxb1)
        semx = (semx0, semx1)
        semi = (semi0, semi1)

        rows_per_tile = G // NS
        _zero_rows(xbuf0, 0, rows_per_tile)
        pltpu.sync_copy(xbuf0.at[pl.ds(0, rows_per_tile)],
                        acc_sh.at[pl.ds(sid * rows_per_tile, rows_per_tile)])
        plsc.subcore_barrier()

        @pl.when(wid < 2)
        def _():
            pltpu.sync_copy(packed_hbm, packed_v)
            lane = lax.broadcasted_iota(jnp.int32, (L,), 0)
            half = wid * (G // 2)

            def cnt_body(t, carry):
                g0 = half + t * L
                lb_lo = _lb_packed(packed_v, g0 + lane)
                lb_hi = _lb_packed(packed_v, g0 + 1 + lane)
                cnt_v[pl.ds(g0, L)] = (lb_hi - lb_lo).astype(jnp.float32)
                return carry

            lax.fori_loop(0, G // 2 // L, cnt_body, 0)
            pltpu.sync_copy(cnt_v.at[pl.ds(half, G // 2)],
                            cnts_hbm.at[pl.ds(half, G // 2)])

        @pl.when(wid >= 2)
        def _():
            j = wid - 2

            def issue(k, b):
                c = j + NSTREAM * k
                di0 = pltpu.async_copy(batch_hbm.at[pl.ds(c * CH, 128)],
                                       idxb[b].at[0], semi[b])
                di1 = pltpu.async_copy(batch_hbm.at[pl.ds(c * CH + 128, 128)],
                                       idxb[b].at[1], semi[b])
                dx = pltpu.async_copy(x_hbm.at[pl.ds(c * CH, CH)], xbuf[b],
                                      semx[b])
                return di0, di1, dx

            descs = {0: issue(0, 0), 1: issue(1, 1)}
            for k in range(KPW):
                b = k & 1
                di0, di1, dx = descs.pop(k)
                di0.wait()
                di1.wait()
                dx.wait()
                for h in range(2):
                    pltpu.sync_copy(xbuf[b].at[pl.ds(h * 128, 128)],
                                    acc_sh.at[idxb[b].at[h]], add=True)
                if k + 2 < KPW:
                    descs[k + 2] = issue(k + 2, b)

        plsc.subcore_barrier()

        lo = sid * rows_per_tile
        pltpu.sync_copy(acc_sh.at[pl.ds(lo, rows_per_tile)],
                        parts_hbm.at[cid, pl.ds(lo, rows_per_tile)])

    return pl.kernel(
        body,
        out_type=(
            jax.ShapeDtypeStruct((NC, G, D), jnp.float32),
            jax.ShapeDtypeStruct((G,), jnp.float32),
        ),
        mesh=mesh,
        scratch_types=[
            pltpu.VMEM((CH, D), jnp.float32),
            pltpu.VMEM((CH, D), jnp.float32),
            pltpu.VMEM((2, 128), jnp.int32),
            pltpu.VMEM((2, 128), jnp.int32),
            pltpu.VMEM((NPACK,), jnp.int32),
            pltpu.VMEM((G,), jnp.float32),
            pltpu.SemaphoreType.DMA,
            pltpu.SemaphoreType.DMA,
            pltpu.SemaphoreType.DMA,
            pltpu.SemaphoreType.DMA,
            pltpu.VMEM_SHARED((G, D), jnp.float32),
        ],
        compiler_params=pltpu.CompilerParams(needs_layout_passes=False),
    )(x, batch, packed)


def _tc_partial_body(xb_ref, bb_ref, o_ref):
    i = pl.program_id(0)

    @pl.when(i == 0)
    def _():
        o_ref[...] = jnp.zeros((G, D), jnp.float32)

    bb = bb_ref[0, 0, :]
    gids = lax.broadcasted_iota(jnp.int32, (G, TB), 0)
    oh = jnp.where(gids == bb[None, :], 1.0, 0.0).astype(jnp.float32)
    o_ref[...] += jnp.dot(oh, xb_ref[...], preferred_element_type=jnp.float32)


def _tc_partial(x, batch3):
    blk0 = SC_ROWS // TB
    return pl.pallas_call(
        _tc_partial_body,
        grid=(TC_BLOCKS,),
        in_specs=[
            pl.BlockSpec((TB, D), lambda i: (blk0 + i, 0)),
            pl.BlockSpec((1, 1, TB), lambda i: (blk0 + i, 0, 0)),
        ],
        out_specs=pl.BlockSpec((G, D), lambda i: (0, 0)),
        out_shape=jax.ShapeDtypeStruct((G, D), jnp.float32),
    )(x, batch3)


def _head_body(parts_ref, tc_ref, cnts_ref, w_ref, b_ref, o_ref):
    sums = parts_ref[0] + parts_ref[1] + tc_ref[...]
    rep = sums / jnp.maximum(cnts_ref[...], 1.0)
    o_ref[...] = (
        jnp.dot(rep, w_ref[...], preferred_element_type=jnp.float32)
        + b_ref[...]
    )


def kernel(x, batch, W, b):
    bi = batch.astype(jnp.int32)
    packed = jnp.bitwise_or(bi[:NPACK], jnp.left_shift(bi[NPACK:], 16))
    batch3 = bi.reshape(N // TB, 1, TB)
    tc_part = _tc_partial(x, batch3)
    parts, cnts = _sc_segment_sums(x, bi, packed)
    out = pl.pallas_call(
        _head_body,
        out_shape=jax.ShapeDtypeStruct((G, T), jnp.float32),
    )(parts, tc_part, cnts.reshape(G, 1), W, b.reshape(1, T))
    return out


# speedup vs baseline: 1.1893x; 1.0064x over previous
import jax
import jax.numpy as jnp
from jax import lax
from jax.experimental import pallas as pl
from jax.experimental.pallas import tpu as pltpu
from jax.experimental.pallas import tpu_sc as plsc

N = 100000
D = 128
G = 512
T = 12

NC = 2
NS = 16
NW = NC * NS
L = 16

CH = 256
NSTREAM = NW - 2
KPW = 10
SC_ROWS = CH * NSTREAM * KPW
TC_ROWS = N - SC_ROWS
TB = 800
TC_BLOCKS = TC_ROWS // TB
BSTEPS = 17
NPACK = N // 2


def _zero_rows(ref, row0, rows):
    z = jnp.zeros((L,), jnp.float32)

    def body(i, carry):
        for j in range(D // L):
            ref[i, pl.ds(j * L, L)] = z
        return carry

    lax.fori_loop(row0, row0 + rows, body, 0)


def _lb_packed(pv, targets):
    half = jnp.full((L,), NPACK, jnp.int32)
    lo0 = jnp.zeros((L,), jnp.int32)
    hi0 = jnp.full((L,), N, jnp.int32)
    nm1 = jnp.full((L,), N - 1, jnp.int32)
    one = jnp.full((L,), 1, jnp.int32)

    def step(_, carry):
        lo, hi = carry
        mid = lax.shift_right_logical(lo + hi, one)
        midc = jnp.minimum(mid, nm1)
        in_lo = midc < half
        word = jnp.where(in_lo, midc, midc - NPACK)
        w = plsc.load_gather(pv, [word])
        sh = jnp.where(in_lo, jnp.zeros((L,), jnp.int32),
                       jnp.full((L,), 16, jnp.int32))
        val = jnp.bitwise_and(lax.shift_right_logical(w, sh),
                              jnp.full((L,), 0xFFFF, jnp.int32))
        pred = val >= targets
        act = lo < hi
        hi = jnp.where(jnp.logical_and(pred, act), midc, hi)
        lo = jnp.where(jnp.logical_and(jnp.logical_not(pred), act),
                       midc + 1, lo)
        return lo, hi

    lo, _ = lax.fori_loop(0, BSTEPS, step, (lo0, hi0))
    return lo


def _sc_segment_sums(x, batch, packed):
    mesh = plsc.VectorSubcoreMesh(core_axis_name="c", subcore_axis_name="s")

    def body(x_hbm, batch_hbm, packed_hbm, parts_hbm, cnts_hbm,
             xbuf0, xbuf1, idxb0, idxb1, packed_v, cnt_v,
             semx0, semx1, semi0, semi1, acc_sh):
        cid = lax.axis_index("c")
        sid = lax.axis_index("s")
        wid = sid * NC + cid
        xbuf = (xbuf0, xbuf1)
        idxb = (idxb0, idxb1)
        semx = (semx0, semx1)
        semi = (semi0, semi1)

        rows_per_tile = G // NS
        _zero_rows(xbuf0, 0, rows_per_tile)
        pltpu.sync_copy(xbuf0.at[pl.ds(0, rows_per_tile)],
                        acc_sh.at[pl.ds(sid * rows_per_tile, rows_per_tile)])
        plsc.subcore_barrier()

        @pl.when(wid < 2)
        def _():
            pltpu.sync_copy(packed_hbm, packed_v)
            lane = lax.broadcasted_iota(jnp.int32, (L,), 0)
            half = wid * (G // 2)

            def cnt_body(t, carry):
                g0 = half + t * L
                lb_lo = _lb_packed(packed_v, g0 + lane)
                lb_hi = _lb_packed(packed_v, g0 + 1 + lane)
                cnt_v[pl.ds(g0, L)] = (lb_hi - lb_lo).astype(jnp.float32)
                return carry

            lax.fori_loop(0, G // 2 // L, cnt_body, 0)
            pltpu.sync_copy(cnt_v.at[pl.ds(half, G // 2)],
                            cnts_hbm.at[pl.ds(half, G // 2)])

        @pl.when(wid >= 2)
        def _():
            j = wid - 2

            def issue(k, b):
                c = j + NSTREAM * k
                pltpu.async_copy(batch_hbm.at[pl.ds(c * CH, 128)],
                                 idxb[b].at[0], semi[b])
                pltpu.async_copy(batch_hbm.at[pl.ds(c * CH + 128, 128)],
                                 idxb[b].at[1], semi[b])
                pltpu.async_copy(x_hbm.at[pl.ds(c * CH, CH)], xbuf[b],
                                 semx[b])

            issue(0, 0)
            issue(1, 1)

            def pair_body(k2, carry):
                for b in range(2):
                    k = 2 * k2 + b
                    for _ in range(2):
                        pltpu.make_async_copy(
                            batch_hbm.at[pl.ds(0, 128)], idxb[b].at[0],
                            semi[b]).wait()
                    pltpu.make_async_copy(
                        x_hbm.at[pl.ds(0, CH)], xbuf[b], semx[b]).wait()
                    for h in range(2):
                        pltpu.sync_copy(xbuf[b].at[pl.ds(h * 128, 128)],
                                        acc_sh.at[idxb[b].at[h]], add=True)

                    @pl.when(k + 2 < KPW)
                    def _():
                        issue(k + 2, b)

                return carry

            lax.fori_loop(0, KPW // 2, pair_body, 0)

        plsc.subcore_barrier()

        lo = sid * rows_per_tile
        pltpu.sync_copy(acc_sh.at[pl.ds(lo, rows_per_tile)],
                        parts_hbm.at[cid, pl.ds(lo, rows_per_tile)])

    return pl.kernel(
        body,
        out_type=(
            jax.ShapeDtypeStruct((NC, G, D), jnp.float32),
            jax.ShapeDtypeStruct((G,), jnp.float32),
        ),
        mesh=mesh,
        scratch_types=[
            pltpu.VMEM((CH, D), jnp.float32),
            pltpu.VMEM((CH, D), jnp.float32),
            pltpu.VMEM((2, 128), jnp.int32),
            pltpu.VMEM((2, 128), jnp.int32),
            pltpu.VMEM((NPACK,), jnp.int32),
            pltpu.VMEM((G,), jnp.float32),
            pltpu.SemaphoreType.DMA,
            pltpu.SemaphoreType.DMA,
            pltpu.SemaphoreType.DMA,
            pltpu.SemaphoreType.DMA,
            pltpu.VMEM_SHARED((G, D), jnp.float32),
        ],
        compiler_params=pltpu.CompilerParams(needs_layout_passes=False),
    )(x, batch, packed)


def _tc_partial_body(xb_ref, bb_ref, o_ref):
    i = pl.program_id(0)

    @pl.when(i == 0)
    def _():
        o_ref[...] = jnp.zeros((G, D), jnp.float32)

    bb = bb_ref[0, 0, :]
    gids = lax.broadcasted_iota(jnp.int32, (G, TB), 0)
    oh = jnp.where(gids == bb[None, :], 1.0, 0.0).astype(jnp.float32)
    o_ref[...] += jnp.dot(oh, xb_ref[...], preferred_element_type=jnp.float32)


def _tc_partial(x, batch3):
    blk0 = SC_ROWS // TB
    return pl.pallas_call(
        _tc_partial_body,
        grid=(TC_BLOCKS,),
        in_specs=[
            pl.BlockSpec((TB, D), lambda i: (blk0 + i, 0)),
            pl.BlockSpec((1, 1, TB), lambda i: (blk0 + i, 0, 0)),
        ],
        out_specs=pl.BlockSpec((G, D), lambda i: (0, 0)),
        out_shape=jax.ShapeDtypeStruct((G, D), jnp.float32),
    )(x, batch3)


def _head_body(parts_ref, tc_ref, cnts_ref, w_ref, b_ref, o_ref):
    sums = parts_ref[0] + parts_ref[1] + tc_ref[...]
    rep = sums / jnp.maximum(cnts_ref[...], 1.0)
    o_ref[...] = (
        jnp.dot(rep, w_ref[...], preferred_element_type=jnp.float32)
        + b_ref[...]
    )


def kernel(x, batch, W, b):
    bi = batch.astype(jnp.int32)
    packed = jnp.bitwise_or(bi[:NPACK], jnp.left_shift(bi[NPACK:], 16))
    batch3 = bi.reshape(N // TB, 1, TB)
    tc_part = _tc_partial(x, batch3)
    parts, cnts = _sc_segment_sums(x, bi, packed)
    out = pl.pallas_call(
        _head_body,
        out_shape=jax.ShapeDtypeStruct((G, T), jnp.float32),
    )(parts, tc_part, cnts.reshape(G, 1), W, b.reshape(1, T))
    return out
